# baseline (speedup 1.0000x reference)
import jax
import jax.numpy as jnp
from jax.experimental import pallas as pl
from jax.experimental.pallas import tpu as pltpu


def _copy_hbm(v_ref, o_ref, sem):
    copy = pltpu.make_async_copy(v_ref, o_ref, sem)
    copy.start()
    copy.wait()


def kernel(video, ques, attr, emb):
    del ques, attr, emb
    out = pl.pallas_call(
        _copy_hbm,
        in_specs=[pl.BlockSpec(memory_space=pl.ANY)],
        out_specs=pl.BlockSpec(memory_space=pl.ANY),
        out_shape=jax.ShapeDtypeStruct(video.shape, video.dtype),
        scratch_shapes=[pltpu.SemaphoreType.DMA],
    )(video)
    return out
